# baseline (speedup 1.0000x reference)
import jax
import jax.numpy as jnp
from jax.experimental import pallas as pl
from jax.experimental.pallas import tpu as pltpu

N = 4096
D = 768
H = 128
NW = 12
NL = 13
PAD = 2048
CHUNK = 128
DBLK = 128
OFFS = [0] + [2 ** k for k in range(NL - 1)]


def _gelu(x):
    return 0.5 * x * (1.0 + jax.lax.erf(x * (2.0 ** -0.5)))


def _w_builder_kernel(inp_ref, w1_ref, b1_ref, w2_ref, b2_ref, out_ref):
    h = _gelu(
        jnp.dot(inp_ref[...], w1_ref[0], preferred_element_type=jnp.float32)
        + b1_ref[0]
    )
    out_ref[0] = (
        jnp.dot(h, w2_ref[0], preferred_element_type=jnp.float32) + b2_ref[0]
    )


def _gmlp_kernel(v_ref, w1_ref, b1_ref, w2_ref, b2_ref, out_ref):
    h = _gelu(
        jnp.dot(v_ref[...], w1_ref[...], preferred_element_type=jnp.float32)
        + b1_ref[...]
    )
    out_ref[...] = (
        jnp.dot(h, w2_ref[...], preferred_element_type=jnp.float32) + b2_ref[...]
    )


def _chord_kernel(vg_ref, w_ref, out_ref, buf, wbuf, sems):
    cp_main = pltpu.make_async_copy(vg_ref, buf.at[0:N, :], sems.at[0])
    cp_tail = pltpu.make_async_copy(vg_ref.at[0:PAD, :], buf.at[N:N + PAD, :],
                                    sems.at[1])
    cp_main.start()
    cp_tail.start()
    cp_main.wait()
    cp_tail.wait()
    pltpu.make_async_copy(w_ref.at[0], wbuf.at[0], sems.at[2]).start()

    nchunks = N // CHUNK

    def body(m, carry):
        slot = jax.lax.rem(m, 2)
        nxt = 1 - slot

        @pl.when(m < NW - 1)
        def _():
            pltpu.make_async_copy(w_ref.at[m + 1], wbuf.at[nxt],
                                  sems.at[2 + nxt]).start()

        pltpu.make_async_copy(w_ref.at[m], wbuf.at[slot],
                              sems.at[2 + slot]).wait()

        def chunk_body(c, _):
            r0 = c * CHUNK
            wv = wbuf[slot, pl.ds(r0, CHUNK), :]
            for d0 in range(0, D, DBLK):
                base = buf[pl.ds(r0, CHUNK + 8), d0:d0 + DBLK]
                acc = base[0:CHUNK] * (wv[:, 0:1] + 1.0)
                for k in range(1, NL):
                    off = OFFS[k]
                    if off < 8:
                        src = base[off:off + CHUNK]
                    else:
                        src = buf[pl.ds(r0 + off, CHUNK), d0:d0 + DBLK]
                    acc = acc + wv[:, k:k + 1] * src
                buf[pl.ds(r0, CHUNK), d0:d0 + DBLK] = acc
            return _

        jax.lax.fori_loop(0, nchunks, chunk_body, 0)

        def tail_body(c, _):
            r0 = c * CHUNK
            buf[pl.ds(N + r0, CHUNK), :] = buf[pl.ds(r0, CHUNK), :]
            return _

        jax.lax.fori_loop(0, PAD // CHUNK, tail_body, 0)
        return carry

    jax.lax.fori_loop(0, NW, body, 0)
    cp_out = pltpu.make_async_copy(buf.at[0:N, :], out_ref, sems.at[0])
    cp_out.start()
    cp_out.wait()


def kernel(V, input, g_W1, g_b1, g_W2, g_b2, fs_W1, fs_b1, fs_W2, fs_b2, rows, cols):
    del rows, cols
    W = pl.pallas_call(
        _w_builder_kernel,
        grid=(NW,),
        in_specs=[
            pl.BlockSpec((N, D), lambda m: (0, 0)),
            pl.BlockSpec((1, D, H), lambda m: (m, 0, 0)),
            pl.BlockSpec((1, 1, H), lambda m: (m, 0, 0)),
            pl.BlockSpec((1, H, NL), lambda m: (m, 0, 0)),
            pl.BlockSpec((1, 1, NL), lambda m: (m, 0, 0)),
        ],
        out_specs=pl.BlockSpec((1, N, NL), lambda m: (m, 0, 0)),
        out_shape=jax.ShapeDtypeStruct((NW, N, NL), jnp.float32),
    )(input, fs_W1, fs_b1.reshape(NW, 1, H), fs_W2, fs_b2.reshape(NW, 1, NL))
    GB = 1024
    Vg = pl.pallas_call(
        _gmlp_kernel,
        grid=(N // GB,),
        in_specs=[
            pl.BlockSpec((GB, D), lambda i: (i, 0)),
            pl.BlockSpec((D, H), lambda i: (0, 0)),
            pl.BlockSpec((1, H), lambda i: (0, 0)),
            pl.BlockSpec((H, D), lambda i: (0, 0)),
            pl.BlockSpec((1, D), lambda i: (0, 0)),
        ],
        out_specs=pl.BlockSpec((GB, D), lambda i: (i, 0)),
        out_shape=jax.ShapeDtypeStruct((N, D), jnp.float32),
    )(V, g_W1, g_b1.reshape(1, H), g_W2, g_b2.reshape(1, D))
    out = pl.pallas_call(
        _chord_kernel,
        in_specs=[
            pl.BlockSpec(memory_space=pl.ANY),
            pl.BlockSpec(memory_space=pl.ANY),
        ],
        out_specs=pl.BlockSpec(memory_space=pl.ANY),
        out_shape=jax.ShapeDtypeStruct((N, D), jnp.float32),
        scratch_shapes=[
            pltpu.VMEM((N + PAD, D), jnp.float32),
            pltpu.VMEM((2, N, NL), jnp.float32),
            pltpu.SemaphoreType.DMA((4,)),
        ],
    )(Vg, W)
    return out


# speedup vs baseline: 1.1997x; 1.1997x over previous
import jax
import jax.numpy as jnp
from jax.experimental import pallas as pl
from jax.experimental.pallas import tpu as pltpu

N = 4096
D = 768
H = 128
NW = 12
NL = 13
PAD = 2048
CHUNK = 512
OFFS = [0] + [2 ** k for k in range(NL - 1)]


def _gelu(x):
    return 0.5 * x * (1.0 + jax.lax.erf(x * (2.0 ** -0.5)))


def _mlp_w(inp_ref, fw1_ref, fb1_ref, fw2_ref, fb2_ref, m):
    h = _gelu(
        jnp.dot(inp_ref[...], fw1_ref[m], preferred_element_type=jnp.float32)
        + fb1_ref[m]
    )
    return jnp.dot(h, fw2_ref[m], preferred_element_type=jnp.float32) + fb2_ref[m]


def _fused_kernel(v_ref, inp_ref, gw1_ref, gb1_ref, gw2_ref, gb2_ref,
                  fw1_ref, fb1_ref, fw2_ref, fb2_ref, out_ref,
                  buf, wbuf, sems):
    cp_in = pltpu.make_async_copy(v_ref, buf.at[0:N, :], sems.at[0])
    cp_in.start()
    cp_in.wait()

    nchunks = N // CHUNK

    def gmlp_body(c, carry):
        r0 = c * CHUNK
        x = buf[pl.ds(r0, CHUNK), :]
        h = _gelu(
            jnp.dot(x, gw1_ref[...], preferred_element_type=jnp.float32)
            + gb1_ref[...]
        )
        y = jnp.dot(h, gw2_ref[...], preferred_element_type=jnp.float32) + gb2_ref[...]
        buf[pl.ds(r0, CHUNK), :] = y

        @pl.when(r0 < PAD)
        def _tail():
            buf[pl.ds(N + r0, CHUNK), :] = y

        return carry

    jax.lax.fori_loop(0, nchunks, gmlp_body, 0)

    wbuf[0] = _mlp_w(inp_ref, fw1_ref, fb1_ref, fw2_ref, fb2_ref, 0)

    def body(m, carry):
        slot = jax.lax.rem(m, 2)
        nxt = 1 - slot

        @pl.when(m < NW - 1)
        def _():
            wbuf[nxt] = _mlp_w(inp_ref, fw1_ref, fb1_ref, fw2_ref, fb2_ref,
                               m + 1)

        def chunk_body(c, _):
            r0 = c * CHUNK
            base = buf[pl.ds(r0, CHUNK + 8), :]
            acc = base[0:CHUNK] * (wbuf[slot, pl.ds(r0, CHUNK), 0:1] + 1.0)
            for k in range(1, NL):
                off = OFFS[k]
                if off < 8:
                    src = base[off:off + CHUNK]
                else:
                    src = buf[pl.ds(r0 + off, CHUNK), :]
                acc = acc + wbuf[slot, pl.ds(r0, CHUNK), k:k + 1] * src
            buf[pl.ds(r0, CHUNK), :] = acc
            return _

        jax.lax.fori_loop(0, nchunks, chunk_body, 0)

        def tail_body(c, _):
            r0 = c * CHUNK
            buf[pl.ds(N + r0, CHUNK), :] = buf[pl.ds(r0, CHUNK), :]
            return _

        jax.lax.fori_loop(0, PAD // CHUNK, tail_body, 0)
        return carry

    jax.lax.fori_loop(0, NW, body, 0)
    cp_out = pltpu.make_async_copy(buf.at[0:N, :], out_ref, sems.at[0])
    cp_out.start()
    cp_out.wait()


def kernel(V, input, g_W1, g_b1, g_W2, g_b2, fs_W1, fs_b1, fs_W2, fs_b2, rows, cols):
    del rows, cols
    out = pl.pallas_call(
        _fused_kernel,
        in_specs=[
            pl.BlockSpec(memory_space=pl.ANY),
            pl.BlockSpec(memory_space=pltpu.VMEM),
            pl.BlockSpec(memory_space=pltpu.VMEM),
            pl.BlockSpec(memory_space=pltpu.VMEM),
            pl.BlockSpec(memory_space=pltpu.VMEM),
            pl.BlockSpec(memory_space=pltpu.VMEM),
            pl.BlockSpec(memory_space=pltpu.VMEM),
            pl.BlockSpec(memory_space=pltpu.VMEM),
            pl.BlockSpec(memory_space=pltpu.VMEM),
            pl.BlockSpec(memory_space=pltpu.VMEM),
        ],
        out_specs=pl.BlockSpec(memory_space=pl.ANY),
        out_shape=jax.ShapeDtypeStruct((N, D), jnp.float32),
        scratch_shapes=[
            pltpu.VMEM((N + PAD, D), jnp.float32),
            pltpu.VMEM((2, N, NL), jnp.float32),
            pltpu.SemaphoreType.DMA((2,)),
        ],
    )(V, input, g_W1, g_b1.reshape(1, H), g_W2, g_b2.reshape(1, D),
      fs_W1, fs_b1.reshape(NW, 1, H), fs_W2, fs_b2.reshape(NW, 1, NL))
    return out
